# Initial kernel scaffold; baseline (speedup 1.0000x reference)
import jax
import jax.numpy as jnp
from jax.experimental import pallas as pl


def kernel(x, w_gate, w_noise, expert_w, expert_b):
    raise NotImplementedError("write your pallas kernel here")


# speedup vs baseline: 1.9458x; 1.9458x over previous
import functools

import jax
import jax.numpy as jnp
from jax import lax
from jax.experimental import pallas as pl
from jax.experimental.pallas import tpu as pltpu

E = 16
K = 2
DIN = 1024
DHID = 4096
N = 8192
M = N * K

TM = 256
NT = M // TM
WMAX = NT + E
TN = 2048
NB = DHID // TN

_NEG = -3e38


def _router_body(x_ref, wg_ref, idx_ref, gate_ref):
    logits = jnp.dot(x_ref[...], wg_ref[...], preferred_element_type=jnp.float32)
    col = lax.broadcasted_iota(jnp.int32, logits.shape, 1)
    m1 = jnp.max(logits, axis=1, keepdims=True)
    i1 = jnp.min(jnp.where(logits == m1, col, E), axis=1, keepdims=True)
    l2 = jnp.where(col == i1, _NEG, logits)
    m2 = jnp.max(l2, axis=1, keepdims=True)
    i2 = jnp.min(jnp.where(l2 == m2, col, E), axis=1, keepdims=True)
    a = jnp.exp(m2 - m1)
    g1 = 1.0 / (1.0 + a)
    idx_ref[...] = jnp.concatenate([i1, i2], axis=1)
    gate_ref[...] = jnp.concatenate([g1, 1.0 - g1], axis=1)


def _router(x, w_gate):
    bm = 512
    return pl.pallas_call(
        _router_body,
        grid=(N // bm,),
        in_specs=[
            pl.BlockSpec((bm, DIN), lambda i: (i, 0)),
            pl.BlockSpec((DIN, E), lambda i: (0, 0)),
        ],
        out_specs=[
            pl.BlockSpec((bm, K), lambda i: (i, 0)),
            pl.BlockSpec((bm, K), lambda i: (i, 0)),
        ],
        out_shape=[
            jax.ShapeDtypeStruct((N, K), jnp.int32),
            jax.ShapeDtypeStruct((N, K), jnp.float32),
        ],
    )(x, w_gate)


def _gmm_body(tile_ref, ex_ref, lo_ref, hi_ref, first_ref,
              xs_ref, w_ref, b_ref, y_ref):
    w = pl.program_id(1)
    lo = lo_ref[w]
    hi = hi_ref[w]

    @pl.when(hi > lo)
    def _():
        acc = jnp.dot(xs_ref[...], w_ref[0], preferred_element_type=jnp.float32)
        acc += b_ref[0]
        rows = tile_ref[w] * TM + lax.broadcasted_iota(jnp.int32, (TM, 1), 0)
        mask = (rows >= lo) & (rows < hi)
        contrib = jnp.where(mask, acc, 0.0)
        y_ref[...] = jnp.where(first_ref[w] > 0, contrib, contrib + y_ref[...])


def _gmm(wl_tile, wl_ex, wl_lo, wl_hi, wl_first, x_sorted, expert_w, expert_b):
    grid_spec = pltpu.PrefetchScalarGridSpec(
        num_scalar_prefetch=5,
        grid=(NB, WMAX),
        in_specs=[
            pl.BlockSpec((TM, DIN), lambda n, w, t, e, lo, hi, f: (t[w], 0)),
            pl.BlockSpec((1, DIN, TN), lambda n, w, t, e, lo, hi, f: (e[w], 0, n)),
            pl.BlockSpec((1, 1, TN), lambda n, w, t, e, lo, hi, f: (e[w], 0, n)),
        ],
        out_specs=pl.BlockSpec((TM, TN), lambda n, w, t, e, lo, hi, f: (t[w], n)),
    )
    return pl.pallas_call(
        _gmm_body,
        grid_spec=grid_spec,
        out_shape=jax.ShapeDtypeStruct((M, DHID), jnp.float32),
        compiler_params=pltpu.CompilerParams(
            dimension_semantics=("arbitrary", "arbitrary")),
    )(wl_tile, wl_ex, wl_lo, wl_hi, wl_first, x_sorted, expert_w, expert_b)


def _combine_body(z_ref, g_ref, o_ref):
    z = z_ref[...]
    g0 = g_ref[:, 0:1]
    g1 = g_ref[:, 1:2]
    o_ref[...] = g0 * z[:, 0, :] + g1 * z[:, 1, :]


def _combine(z, gates):
    bm = 256
    return pl.pallas_call(
        _combine_body,
        grid=(N // bm,),
        in_specs=[
            pl.BlockSpec((bm, K, DHID), lambda i: (i, 0, 0)),
            pl.BlockSpec((bm, K), lambda i: (i, 0)),
        ],
        out_specs=pl.BlockSpec((bm, DHID), lambda i: (i, 0)),
        out_shape=jax.ShapeDtypeStruct((N, DHID), jnp.float32),
    )(z, gates)


def _dispatch_jnp(e_flat):
    order = jnp.argsort(e_flat, stable=True).astype(jnp.int32)
    pos = jnp.argsort(order).astype(jnp.int32)
    tid_sorted = order // K
    counts = jnp.bincount(e_flat, length=E).astype(jnp.int32)
    ends = jnp.cumsum(counts)
    offs = ends - counts
    t0 = offs // TM
    t1 = jnp.where(counts > 0, (ends + TM - 1) // TM, t0)
    nt = t1 - t0
    base = jnp.cumsum(nt) - nt
    wl_tile = jnp.full((WMAX,), NT - 1, jnp.int32)
    wl_ex = jnp.zeros((WMAX,), jnp.int32)
    wl_lo = jnp.zeros((WMAX,), jnp.int32)
    wl_hi = jnp.zeros((WMAX,), jnp.int32)
    wl_first = jnp.zeros((WMAX,), jnp.int32)
    for s in range(NT):
        m = s < nt
        widx = jnp.where(m, base + s, WMAX - 1)
        tile = t0 + s
        lo = jnp.maximum(offs, tile * TM)
        hi = jnp.minimum(ends, (tile + 1) * TM)
        wl_tile = wl_tile.at[widx].set(jnp.where(m, tile, wl_tile[widx]))
        wl_ex = wl_ex.at[widx].set(jnp.where(m, jnp.arange(E, dtype=jnp.int32), wl_ex[widx]))
        wl_lo = wl_lo.at[widx].set(jnp.where(m, lo, wl_lo[widx]))
        wl_hi = wl_hi.at[widx].set(jnp.where(m, hi, wl_hi[widx]))
        wl_first = wl_first.at[widx].set(jnp.where(m, (lo == tile * TM).astype(jnp.int32), wl_first[widx]))
    return tid_sorted, pos, (wl_tile, wl_ex, wl_lo, wl_hi, wl_first)


def kernel(x, w_gate, w_noise, expert_w, expert_b):
    idx, gates = _router(x, w_gate)
    e_flat = idx.reshape(M)
    tid_sorted, pos, wl = _dispatch_jnp(e_flat)
    x_sorted = jnp.take(x, tid_sorted, axis=0)
    y = _gmm(*wl, x_sorted, expert_w, expert_b.reshape(E, 1, DHID))
    z = jnp.take(y, pos, axis=0).reshape(N, K, DHID)
    return _combine(z, gates)


# speedup vs baseline: 2.4039x; 1.2355x over previous
import functools

import jax
import jax.numpy as jnp
from jax import lax
from jax.experimental import pallas as pl
from jax.experimental.pallas import tpu as pltpu
from jax.experimental.pallas import tpu_sc as plsc

E = 16
K = 2
DIN = 1024
DHID = 4096
N = 8192
M = N * K

TM = 256
NT = M // TM
WMAX = NT + E
TN = 2048
NB = DHID // TN

_NEG = -3e38


def _router_body(x_ref, wg_ref, idx_ref, gate_ref):
    logits = jnp.dot(x_ref[...], wg_ref[...], preferred_element_type=jnp.float32)
    col = lax.broadcasted_iota(jnp.int32, logits.shape, 1)
    m1 = jnp.max(logits, axis=1, keepdims=True)
    i1 = jnp.min(jnp.where(logits == m1, col, E), axis=1, keepdims=True)
    l2 = jnp.where(col == i1, _NEG, logits)
    m2 = jnp.max(l2, axis=1, keepdims=True)
    i2 = jnp.min(jnp.where(l2 == m2, col, E), axis=1, keepdims=True)
    a = jnp.exp(m2 - m1)
    g1 = 1.0 / (1.0 + a)
    idx_ref[...] = jnp.concatenate([i1, i2], axis=1)
    gate_ref[...] = jnp.concatenate([g1, 1.0 - g1], axis=1)


def _router(x, w_gate):
    bm = 512
    return pl.pallas_call(
        _router_body,
        grid=(N // bm,),
        in_specs=[
            pl.BlockSpec((bm, DIN), lambda i: (i, 0)),
            pl.BlockSpec((DIN, E), lambda i: (0, 0)),
        ],
        out_specs=[
            pl.BlockSpec((bm, K), lambda i: (i, 0)),
            pl.BlockSpec((bm, K), lambda i: (i, 0)),
        ],
        out_shape=[
            jax.ShapeDtypeStruct((N, K), jnp.int32),
            jax.ShapeDtypeStruct((N, K), jnp.float32),
        ],
    )(x, w_gate)


def _gmm_body(tile_ref, ex_ref, lo_ref, hi_ref, first_ref,
              xs_ref, w_ref, b_ref, y_ref):
    w = pl.program_id(1)
    lo = lo_ref[w]
    hi = hi_ref[w]

    @pl.when(hi > lo)
    def _():
        acc = jnp.dot(xs_ref[...], w_ref[0], preferred_element_type=jnp.float32)
        acc += b_ref[0]
        rows = tile_ref[w] * TM + lax.broadcasted_iota(jnp.int32, (TM, 1), 0)
        mask = (rows >= lo) & (rows < hi)
        contrib = jnp.where(mask, acc, 0.0)
        y_ref[...] = jnp.where(first_ref[w] > 0, contrib, contrib + y_ref[...])


def _gmm(wl_tile, wl_ex, wl_lo, wl_hi, wl_first, x_sorted, expert_w, expert_b):
    grid_spec = pltpu.PrefetchScalarGridSpec(
        num_scalar_prefetch=5,
        grid=(NB, WMAX),
        in_specs=[
            pl.BlockSpec((TM, DIN), lambda n, w, t, e, lo, hi, f: (t[w], 0)),
            pl.BlockSpec((1, DIN, TN), lambda n, w, t, e, lo, hi, f: (e[w], 0, n)),
            pl.BlockSpec((1, 1, TN), lambda n, w, t, e, lo, hi, f: (e[w], 0, n)),
        ],
        out_specs=pl.BlockSpec((TM, TN), lambda n, w, t, e, lo, hi, f: (t[w], n)),
    )
    return pl.pallas_call(
        _gmm_body,
        grid_spec=grid_spec,
        out_shape=jax.ShapeDtypeStruct((M, DHID), jnp.float32),
        compiler_params=pltpu.CompilerParams(
            dimension_semantics=("arbitrary", "arbitrary")),
    )(wl_tile, wl_ex, wl_lo, wl_hi, wl_first, x_sorted, expert_w, expert_b)


def _combine_body(z_ref, g_ref, o_ref):
    z = z_ref[...]
    g0 = g_ref[:, 0:1]
    g1 = g_ref[:, 1:2]
    o_ref[...] = g0 * z[:, 0, :] + g1 * z[:, 1, :]


def _combine(z, gates):
    bm = 256
    return pl.pallas_call(
        _combine_body,
        grid=(N // bm,),
        in_specs=[
            pl.BlockSpec((bm, K, DHID), lambda i: (i, 0, 0)),
            pl.BlockSpec((bm, K), lambda i: (i, 0)),
        ],
        out_specs=pl.BlockSpec((bm, DHID), lambda i: (i, 0)),
        out_shape=jax.ShapeDtypeStruct((N, DHID), jnp.float32),
    )(z, gates)


_NC = 2
_NS = 16
_NW = _NC * _NS


def _make_sc_gather(v, d, b, chunk, dtype=jnp.float32):
    per_w = b // _NW
    n_chunks = per_w // chunk
    mesh = plsc.VectorSubcoreMesh(core_axis_name="c", subcore_axis_name="s")

    @functools.partial(
        pl.kernel, mesh=mesh,
        out_type=jax.ShapeDtypeStruct((b, d), dtype),
        scratch_types=[
            pltpu.VMEM((chunk,), jnp.int32),
            pltpu.VMEM((chunk, d), dtype),
            pltpu.SemaphoreType.DMA,
        ],
    )
    def gk(table_hbm, idx_hbm, out_hbm, idx_v, rows_v, sem):
        wid = lax.axis_index("s") * _NC + lax.axis_index("c")

        def body(ci, _):
            base = wid * per_w + ci * chunk
            pltpu.sync_copy(idx_hbm.at[pl.ds(base, chunk)], idx_v)
            pltpu.async_copy(table_hbm.at[idx_v], rows_v, sem).wait()
            pltpu.sync_copy(rows_v, out_hbm.at[pl.ds(base, chunk)])
            return 0

        lax.fori_loop(0, n_chunks, body, 0)

    return gk


def _dispatch_jnp(e_flat):
    order = jnp.argsort(e_flat, stable=True).astype(jnp.int32)
    pos = jnp.argsort(order).astype(jnp.int32)
    tid_sorted = order // K
    counts = jnp.bincount(e_flat, length=E).astype(jnp.int32)
    ends = jnp.cumsum(counts)
    offs = ends - counts
    t0 = offs // TM
    t1 = jnp.where(counts > 0, (ends + TM - 1) // TM, t0)
    nt = t1 - t0
    base = jnp.cumsum(nt) - nt
    wl_tile = jnp.full((WMAX,), NT - 1, jnp.int32)
    wl_ex = jnp.zeros((WMAX,), jnp.int32)
    wl_lo = jnp.zeros((WMAX,), jnp.int32)
    wl_hi = jnp.zeros((WMAX,), jnp.int32)
    wl_first = jnp.zeros((WMAX,), jnp.int32)
    for s in range(NT):
        m = s < nt
        widx = jnp.where(m, base + s, WMAX - 1)
        tile = t0 + s
        lo = jnp.maximum(offs, tile * TM)
        hi = jnp.minimum(ends, (tile + 1) * TM)
        wl_tile = wl_tile.at[widx].set(jnp.where(m, tile, wl_tile[widx]))
        wl_ex = wl_ex.at[widx].set(jnp.where(m, jnp.arange(E, dtype=jnp.int32), wl_ex[widx]))
        wl_lo = wl_lo.at[widx].set(jnp.where(m, lo, wl_lo[widx]))
        wl_hi = wl_hi.at[widx].set(jnp.where(m, hi, wl_hi[widx]))
        wl_first = wl_first.at[widx].set(jnp.where(m, (lo == tile * TM).astype(jnp.int32), wl_first[widx]))
    return tid_sorted, pos, (wl_tile, wl_ex, wl_lo, wl_hi, wl_first)


def kernel(x, w_gate, w_noise, expert_w, expert_b):
    idx, gates = _router(x, w_gate)
    e_flat = idx.reshape(M)
    tid_sorted, pos, wl = _dispatch_jnp(e_flat)
    x_sorted = _make_sc_gather(N, DIN, M, 64)(x, tid_sorted)
    y = _gmm(*wl, x_sorted, expert_w, expert_b.reshape(E, 1, DHID))
    z = _make_sc_gather(M, DHID, M, 16)(y, pos).reshape(N, K, DHID)
    return _combine(z, gates)


# speedup vs baseline: 2.9601x; 1.2313x over previous
import functools

import jax
import jax.numpy as jnp
from jax import lax
from jax.experimental import pallas as pl
from jax.experimental.pallas import tpu as pltpu
from jax.experimental.pallas import tpu_sc as plsc

E = 16
K = 2
DIN = 1024
DHID = 4096
N = 8192
M = N * K

TM = 256
NT = M // TM
WMAX = NT + E
TN = 2048
NB = DHID // TN

_NEG = -3e38


def _router_body(x_ref, wg_ref, idx_ref, gate_ref):
    logits = jnp.dot(x_ref[...], wg_ref[...], preferred_element_type=jnp.float32)
    col = lax.broadcasted_iota(jnp.int32, logits.shape, 1)
    m1 = jnp.max(logits, axis=1, keepdims=True)
    i1 = jnp.min(jnp.where(logits == m1, col, E), axis=1, keepdims=True)
    l2 = jnp.where(col == i1, _NEG, logits)
    m2 = jnp.max(l2, axis=1, keepdims=True)
    i2 = jnp.min(jnp.where(l2 == m2, col, E), axis=1, keepdims=True)
    a = jnp.exp(m2 - m1)
    g1 = 1.0 / (1.0 + a)
    idx_ref[...] = jnp.concatenate([i1, i2], axis=1)
    gate_ref[...] = jnp.concatenate([g1, 1.0 - g1], axis=1)


def _router(x, w_gate):
    bm = 512
    return pl.pallas_call(
        _router_body,
        grid=(N // bm,),
        in_specs=[
            pl.BlockSpec((bm, DIN), lambda i: (i, 0)),
            pl.BlockSpec((DIN, E), lambda i: (0, 0)),
        ],
        out_specs=[
            pl.BlockSpec((bm, K), lambda i: (i, 0)),
            pl.BlockSpec((bm, K), lambda i: (i, 0)),
        ],
        out_shape=[
            jax.ShapeDtypeStruct((N, K), jnp.int32),
            jax.ShapeDtypeStruct((N, K), jnp.float32),
        ],
    )(x, w_gate)


def _gmm_body(tile_ref, ex_ref, lo_ref, hi_ref, first_ref,
              xs_ref, w_ref, b_ref, y_ref):
    w = pl.program_id(1)
    lo = lo_ref[w]
    hi = hi_ref[w]

    @pl.when(hi > lo)
    def _():
        acc = jnp.dot(xs_ref[...], w_ref[0], preferred_element_type=jnp.float32)
        acc += b_ref[0]
        rows = tile_ref[w] * TM + lax.broadcasted_iota(jnp.int32, (TM, 1), 0)
        mask = (rows >= lo) & (rows < hi)
        contrib = jnp.where(mask, acc, 0.0)
        y_ref[...] = jnp.where(first_ref[w] > 0, contrib, contrib + y_ref[...])


def _gmm(wl_tile, wl_ex, wl_lo, wl_hi, wl_first, x_sorted, expert_w, expert_b):
    grid_spec = pltpu.PrefetchScalarGridSpec(
        num_scalar_prefetch=5,
        grid=(NB, WMAX),
        in_specs=[
            pl.BlockSpec((TM, DIN), lambda n, w, t, e, lo, hi, f: (t[w], 0)),
            pl.BlockSpec((1, DIN, TN), lambda n, w, t, e, lo, hi, f: (e[w], 0, n)),
            pl.BlockSpec((1, 1, TN), lambda n, w, t, e, lo, hi, f: (e[w], 0, n)),
        ],
        out_specs=pl.BlockSpec((TM, TN), lambda n, w, t, e, lo, hi, f: (t[w], n)),
    )
    return pl.pallas_call(
        _gmm_body,
        grid_spec=grid_spec,
        out_shape=jax.ShapeDtypeStruct((M, DHID), jnp.float32),
        compiler_params=pltpu.CompilerParams(
            dimension_semantics=("arbitrary", "arbitrary")),
    )(wl_tile, wl_ex, wl_lo, wl_hi, wl_first, x_sorted, expert_w, expert_b)


def _combine_body(z_ref, g_ref, o_ref):
    z = z_ref[...]
    g0 = g_ref[:, 0:1]
    g1 = g_ref[:, 1:2]
    o_ref[...] = g0 * z[:, 0, :] + g1 * z[:, 1, :]


def _combine(z, gates):
    bm = 256
    return pl.pallas_call(
        _combine_body,
        grid=(N // bm,),
        in_specs=[
            pl.BlockSpec((bm, K, DHID), lambda i: (i, 0, 0)),
            pl.BlockSpec((bm, K), lambda i: (i, 0)),
        ],
        out_specs=pl.BlockSpec((bm, DHID), lambda i: (i, 0)),
        out_shape=jax.ShapeDtypeStruct((N, DHID), jnp.float32),
    )(z, gates)


_NC = 2
_NS = 16
_NW = _NC * _NS


def _make_sc_gather(v, d, b, chunk, dtype=jnp.float32):
    per_w = b // _NW
    n_chunks = per_w // chunk
    mesh = plsc.VectorSubcoreMesh(core_axis_name="c", subcore_axis_name="s")

    @functools.partial(
        pl.kernel, mesh=mesh,
        out_type=jax.ShapeDtypeStruct((b, d), dtype),
        scratch_types=[
            pltpu.VMEM((chunk,), jnp.int32),
            pltpu.VMEM((chunk, d), dtype),
            pltpu.SemaphoreType.DMA,
        ],
    )
    def gk(table_hbm, idx_hbm, out_hbm, idx_v, rows_v, sem):
        wid = lax.axis_index("s") * _NC + lax.axis_index("c")

        def body(ci, _):
            base = wid * per_w + ci * chunk
            pltpu.sync_copy(idx_hbm.at[pl.ds(base, chunk)], idx_v)
            pltpu.async_copy(table_hbm.at[idx_v], rows_v, sem).wait()
            pltpu.sync_copy(rows_v, out_hbm.at[pl.ds(base, chunk)])
            return 0

        lax.fori_loop(0, n_chunks, body, 0)

    return gk


_DC = M // _NS
_DV = _DC // 16


def _dispatch_sc(e_flat):
    mesh = plsc.VectorSubcoreMesh(core_axis_name="c", subcore_axis_name="s")
    i32 = jnp.int32

    @functools.partial(
        pl.kernel, mesh=mesh,
        out_type=[
            jax.ShapeDtypeStruct((M,), i32),
            jax.ShapeDtypeStruct((N,), i32),
            jax.ShapeDtypeStruct((N,), i32),
            jax.ShapeDtypeStruct((WMAX,), i32),
            jax.ShapeDtypeStruct((WMAX,), i32),
            jax.ShapeDtypeStruct((WMAX,), i32),
            jax.ShapeDtypeStruct((WMAX,), i32),
            jax.ShapeDtypeStruct((WMAX,), i32),
            jax.ShapeDtypeStruct((_NS, 16), i32),
        ],
        scratch_types=[
            pltpu.VMEM((_DC,), i32),
            pltpu.VMEM((_DC,), i32),
            pltpu.VMEM((_DC // 2,), i32),
            pltpu.VMEM((_DC // 2,), i32),
            pltpu.VMEM((16,), i32),
            pltpu.VMEM((16, 16), i32),
            pltpu.VMEM((WMAX,), i32),
            pltpu.VMEM((WMAX,), i32),
            pltpu.VMEM((WMAX,), i32),
            pltpu.VMEM((WMAX,), i32),
            pltpu.VMEM((WMAX,), i32),
            pltpu.SMEM((1024,), i32),
        ],
    )
    def dk(e_hbm, pos_hbm, pose_hbm, poso_hbm,
           wt_hbm, we_hbm, wlo_hbm, whi_hbm, wf_hbm, cnt_pub,
           ec, posc, pose_v, poso_v, cntv, ctab,
           wt_s, we_s, wlo_s, whi_s, wf_s, smem):
        cid = lax.axis_index("c")
        sid = lax.axis_index("s")

        lane = lax.iota(i32, 16)
        base = sid * _DC

        @pl.when(cid == 0)
        def _():
            pltpu.sync_copy(e_hbm.at[pl.ds(base, _DC)], ec)

            for b in range(16):
                smem[b] = i32(0)

            def ha(i, c):
                v = ec[pl.ds(i * 16, 16)]
                for l in range(16):
                    e_s = v[l]
                    smem[e_s] = smem[e_s] + 1
                return c

            lax.fori_loop(0, _DV, ha, i32(0))
            cv = jnp.zeros((16,), i32)
            for b in range(16):
                cv = jnp.where(lane == i32(b), smem[b], cv)
            cntv[...] = cv
            pltpu.sync_copy(cntv, cnt_pub.at[sid])

        plsc.subcore_barrier()

        @pl.when(cid == 0)
        def _():
            pltpu.sync_copy(cnt_pub, ctab)
            tot = [i32(0)] * 16
            pre = [i32(0)] * 16
            for w in range(16):
                row = ctab[w, :]
                before = (i32(w) < sid).astype(i32)
                for b in range(16):
                    c = row[b]
                    tot[b] = tot[b] + c
                    pre[b] = pre[b] + c * before
            run = i32(0)
            for b in range(16):
                smem[64 + b] = run
                smem[32 + b] = tot[b]
                smem[b] = run + pre[b]
                run = run + tot[b]

            def hc(g, c):
                v0 = ec[pl.ds(g * 32, 16)]
                v1 = ec[pl.ds(g * 32 + 16, 16)]
                pv0 = jnp.zeros((16,), i32)
                pv1 = jnp.zeros((16,), i32)
                pe = jnp.zeros((16,), i32)
                po = jnp.zeros((16,), i32)
                for l in range(32):
                    e_s = v0[l] if l < 16 else v1[l - 16]
                    p = smem[e_s]
                    smem[e_s] = p + 1
                    if l < 16:
                        pv0 = jnp.where(lane == i32(l), p, pv0)
                    else:
                        pv1 = jnp.where(lane == i32(l - 16), p, pv1)
                    if l % 2 == 0:
                        pe = jnp.where(lane == i32(l // 2), p, pe)
                    else:
                        po = jnp.where(lane == i32(l // 2), p, po)
                posc[pl.ds(g * 32, 16)] = pv0
                posc[pl.ds(g * 32 + 16, 16)] = pv1
                pose_v[pl.ds(g * 16, 16)] = pe
                poso_v[pl.ds(g * 16, 16)] = po
                return c

            lax.fori_loop(0, _DC // 32, hc, i32(0))
            pltpu.sync_copy(posc, pos_hbm.at[pl.ds(base, _DC)])
            pltpu.sync_copy(pose_v, pose_hbm.at[pl.ds(sid * (_DC // 2), _DC // 2)])
            pltpu.sync_copy(poso_v, poso_hbm.at[pl.ds(sid * (_DC // 2), _DC // 2)])

            @pl.when(sid == 0)
            def _():
                for w in range(WMAX):
                    smem[128 + w] = i32(NT - 1)
                    smem[208 + w] = i32(0)
                    smem[288 + w] = i32(0)
                    smem[368 + w] = i32(0)
                    smem[448 + w] = i32(0)
                wcnt = i32(0)
                for e in range(16):
                    off = smem[64 + e]
                    t_ = smem[32 + e]
                    end_ = off + t_
                    t0 = lax.shift_right_logical(off, 8)
                    t1 = jnp.where(t_ > 0,
                                   lax.shift_right_logical(end_ + (TM - 1), 8),
                                   t0)

                    def wb(t, wc, _e=e, _off=off, _end=end_):
                        lo = jnp.maximum(_off, t * TM)
                        hi = jnp.minimum(_end, (t + 1) * TM)
                        fi = (lo == t * TM).astype(i32)
                        smem[128 + wc] = t
                        smem[208 + wc] = i32(_e)
                        smem[288 + wc] = lo
                        smem[368 + wc] = hi
                        smem[448 + wc] = fi
                        return wc + 1

                    wcnt = lax.fori_loop(t0, t1, wb, wcnt)
                for refs, sbase in ((wt_s, 128), (we_s, 208), (wlo_s, 288),
                                    (whi_s, 368), (wf_s, 448)):
                    for j in range(WMAX // 16):
                        v = jnp.zeros((16,), i32)
                        for l in range(16):
                            v = jnp.where(lane == i32(l),
                                          smem[sbase + j * 16 + l], v)
                        refs[pl.ds(j * 16, 16)] = v
                pltpu.sync_copy(wt_s, wt_hbm)
                pltpu.sync_copy(we_s, we_hbm)
                pltpu.sync_copy(wlo_s, wlo_hbm)
                pltpu.sync_copy(whi_s, whi_hbm)
                pltpu.sync_copy(wf_s, wf_hbm)

    p, pe_, po_, wt_, we_, wlo_, whi_, wf_, _ = dk(e_flat)
    return p, pe_, po_, (wt_, we_, wlo_, whi_, wf_)


def _scatter_x(x, pose2, poso2):
    mesh = plsc.VectorSubcoreMesh(core_axis_name="c", subcore_axis_name="s")
    rows_w = N // _NW
    chunk = 32
    n_chunks = rows_w // chunk

    @functools.partial(
        pl.kernel, mesh=mesh,
        out_type=jax.ShapeDtypeStruct((M, DIN), jnp.float32),
        scratch_types=[
            pltpu.VMEM((chunk, DIN), jnp.float32),
            pltpu.VMEM((n_chunks, chunk), jnp.int32),
            pltpu.VMEM((n_chunks, chunk), jnp.int32),
            pltpu.SemaphoreType.DMA,
        ],
    )
    def sk(x_hbm, pe_hbm, po_hbm, xs_hbm, xbuf, pev, pov, sem):
        wid = lax.axis_index("s") * _NC + lax.axis_index("c")
        pltpu.sync_copy(pe_hbm.at[pl.ds(wid * n_chunks, n_chunks)], pev)
        pltpu.sync_copy(po_hbm.at[pl.ds(wid * n_chunks, n_chunks)], pov)

        def body(ci, _):
            trow = wid * rows_w + ci * chunk
            pltpu.sync_copy(x_hbm.at[pl.ds(trow, chunk)], xbuf)
            pltpu.async_copy(xbuf, xs_hbm.at[pev.at[ci]], sem).wait()
            pltpu.async_copy(xbuf, xs_hbm.at[pov.at[ci]], sem).wait()
            return 0

        lax.fori_loop(0, n_chunks, body, 0)

    return sk(x, pose2, poso2)


def _dispatch_jnp(e_flat):
    order = jnp.argsort(e_flat, stable=True).astype(jnp.int32)
    pos = jnp.argsort(order).astype(jnp.int32)
    tid_sorted = order // K
    counts = jnp.bincount(e_flat, length=E).astype(jnp.int32)
    ends = jnp.cumsum(counts)
    offs = ends - counts
    t0 = offs // TM
    t1 = jnp.where(counts > 0, (ends + TM - 1) // TM, t0)
    nt = t1 - t0
    base = jnp.cumsum(nt) - nt
    wl_tile = jnp.full((WMAX,), NT - 1, jnp.int32)
    wl_ex = jnp.zeros((WMAX,), jnp.int32)
    wl_lo = jnp.zeros((WMAX,), jnp.int32)
    wl_hi = jnp.zeros((WMAX,), jnp.int32)
    wl_first = jnp.zeros((WMAX,), jnp.int32)
    for s in range(NT):
        m = s < nt
        widx = jnp.where(m, base + s, WMAX - 1)
        tile = t0 + s
        lo = jnp.maximum(offs, tile * TM)
        hi = jnp.minimum(ends, (tile + 1) * TM)
        wl_tile = wl_tile.at[widx].set(jnp.where(m, tile, wl_tile[widx]))
        wl_ex = wl_ex.at[widx].set(jnp.where(m, jnp.arange(E, dtype=jnp.int32), wl_ex[widx]))
        wl_lo = wl_lo.at[widx].set(jnp.where(m, lo, wl_lo[widx]))
        wl_hi = wl_hi.at[widx].set(jnp.where(m, hi, wl_hi[widx]))
        wl_first = wl_first.at[widx].set(jnp.where(m, (lo == tile * TM).astype(jnp.int32), wl_first[widx]))
    return tid_sorted, pos, (wl_tile, wl_ex, wl_lo, wl_hi, wl_first)


def kernel(x, w_gate, w_noise, expert_w, expert_b):
    idx, gates = _router(x, w_gate)
    e_flat = idx.reshape(M)
    pos, pose, poso, wl = _dispatch_sc(e_flat)
    x_sorted = _scatter_x(x, pose.reshape(N // 32, 32), poso.reshape(N // 32, 32))
    y = _gmm(*wl, x_sorted, expert_w, expert_b.reshape(E, 1, DHID))
    z = _make_sc_gather(M, DHID, M, 16)(y, pos).reshape(N, K, DHID)
    return _combine(z, gates)


# speedup vs baseline: 3.0983x; 1.0467x over previous
import functools

import jax
import jax.numpy as jnp
from jax import lax
from jax.experimental import pallas as pl
from jax.experimental.pallas import tpu as pltpu
from jax.experimental.pallas import tpu_sc as plsc

E = 16
K = 2
DIN = 1024
DHID = 4096
N = 8192
M = N * K

TM = 256
NT = M // TM
WMAX = NT + E
TN = 4096
NB = DHID // TN

_NEG = -3e38


def _router_body(x_ref, wg_ref, idx_ref, gate_ref):
    logits = jnp.dot(x_ref[...], wg_ref[...], preferred_element_type=jnp.float32)
    col = lax.broadcasted_iota(jnp.int32, logits.shape, 1)
    m1 = jnp.max(logits, axis=1, keepdims=True)
    i1 = jnp.min(jnp.where(logits == m1, col, E), axis=1, keepdims=True)
    l2 = jnp.where(col == i1, _NEG, logits)
    m2 = jnp.max(l2, axis=1, keepdims=True)
    i2 = jnp.min(jnp.where(l2 == m2, col, E), axis=1, keepdims=True)
    a = jnp.exp(m2 - m1)
    g1 = 1.0 / (1.0 + a)
    idx_ref[...] = jnp.concatenate([i1, i2], axis=1)
    gate_ref[...] = jnp.concatenate([g1, 1.0 - g1], axis=1)


def _router(x, w_gate):
    bm = 512
    return pl.pallas_call(
        _router_body,
        grid=(N // bm,),
        in_specs=[
            pl.BlockSpec((bm, DIN), lambda i: (i, 0)),
            pl.BlockSpec((DIN, E), lambda i: (0, 0)),
        ],
        out_specs=[
            pl.BlockSpec((bm, K), lambda i: (i, 0)),
            pl.BlockSpec((bm, K), lambda i: (i, 0)),
        ],
        out_shape=[
            jax.ShapeDtypeStruct((N, K), jnp.int32),
            jax.ShapeDtypeStruct((N, K), jnp.float32),
        ],
    )(x, w_gate)


def _gmm_body(tile_ref, ex_ref, lo_ref, hi_ref, first_ref,
              xs_ref, w_ref, b_ref, y_ref):
    w = pl.program_id(1)
    lo = lo_ref[w]
    hi = hi_ref[w]

    @pl.when(hi > lo)
    def _():
        acc = jnp.dot(xs_ref[...], w_ref[0], preferred_element_type=jnp.float32)
        acc += b_ref[0]
        rows = tile_ref[w] * TM + lax.broadcasted_iota(jnp.int32, (TM, 1), 0)
        mask = (rows >= lo) & (rows < hi)
        contrib = jnp.where(mask, acc, 0.0)
        y_ref[...] = jnp.where(first_ref[w] > 0, contrib, contrib + y_ref[...])


def _gmm(wl_tile, wl_ex, wl_lo, wl_hi, wl_first, x_sorted, expert_w, expert_b):
    grid_spec = pltpu.PrefetchScalarGridSpec(
        num_scalar_prefetch=5,
        grid=(NB, WMAX),
        in_specs=[
            pl.BlockSpec((TM, DIN), lambda n, w, t, e, lo, hi, f: (t[w], 0)),
            pl.BlockSpec((1, DIN, TN), lambda n, w, t, e, lo, hi, f: (e[w], 0, n)),
            pl.BlockSpec((1, 1, TN), lambda n, w, t, e, lo, hi, f: (e[w], 0, n)),
        ],
        out_specs=pl.BlockSpec((TM, TN), lambda n, w, t, e, lo, hi, f: (t[w], n)),
    )
    return pl.pallas_call(
        _gmm_body,
        grid_spec=grid_spec,
        out_shape=jax.ShapeDtypeStruct((M, DHID), jnp.float32),
        compiler_params=pltpu.CompilerParams(
            dimension_semantics=("arbitrary", "arbitrary")),
    )(wl_tile, wl_ex, wl_lo, wl_hi, wl_first, x_sorted, expert_w, expert_b)


def _combine_body(z_ref, g_ref, o_ref):
    z = z_ref[...]
    g0 = g_ref[:, 0:1]
    g1 = g_ref[:, 1:2]
    o_ref[...] = g0 * z[:, 0, :] + g1 * z[:, 1, :]


def _combine(z, gates):
    bm = 256
    return pl.pallas_call(
        _combine_body,
        grid=(N // bm,),
        in_specs=[
            pl.BlockSpec((bm, K, DHID), lambda i: (i, 0, 0)),
            pl.BlockSpec((bm, K), lambda i: (i, 0)),
        ],
        out_specs=pl.BlockSpec((bm, DHID), lambda i: (i, 0)),
        out_shape=jax.ShapeDtypeStruct((N, DHID), jnp.float32),
    )(z, gates)


_NC = 2
_NS = 16
_NW = _NC * _NS


def _make_sc_gather(v, d, b, chunk, dtype=jnp.float32):
    per_w = b // _NW
    n_chunks = per_w // chunk
    mesh = plsc.VectorSubcoreMesh(core_axis_name="c", subcore_axis_name="s")

    @functools.partial(
        pl.kernel, mesh=mesh,
        out_type=jax.ShapeDtypeStruct((b, d), dtype),
        scratch_types=[
            pltpu.VMEM((chunk,), jnp.int32),
            pltpu.VMEM((chunk, d), dtype),
            pltpu.SemaphoreType.DMA,
        ],
    )
    def gk(table_hbm, idx_hbm, out_hbm, idx_v, rows_v, sem):
        wid = lax.axis_index("s") * _NC + lax.axis_index("c")

        def body(ci, _):
            base = wid * per_w + ci * chunk
            pltpu.sync_copy(idx_hbm.at[pl.ds(base, chunk)], idx_v)
            pltpu.async_copy(table_hbm.at[idx_v], rows_v, sem).wait()
            pltpu.sync_copy(rows_v, out_hbm.at[pl.ds(base, chunk)])
            return 0

        lax.fori_loop(0, n_chunks, body, 0)

    return gk


_DC = M // _NS
_DV = _DC // 16


def _dispatch_sc(e_flat):
    mesh = plsc.VectorSubcoreMesh(core_axis_name="c", subcore_axis_name="s")
    i32 = jnp.int32

    @functools.partial(
        pl.kernel, mesh=mesh,
        out_type=[
            jax.ShapeDtypeStruct((M,), i32),
            jax.ShapeDtypeStruct((N,), i32),
            jax.ShapeDtypeStruct((N,), i32),
            jax.ShapeDtypeStruct((WMAX,), i32),
            jax.ShapeDtypeStruct((WMAX,), i32),
            jax.ShapeDtypeStruct((WMAX,), i32),
            jax.ShapeDtypeStruct((WMAX,), i32),
            jax.ShapeDtypeStruct((WMAX,), i32),
            jax.ShapeDtypeStruct((_NS, 16), i32),
        ],
        scratch_types=[
            pltpu.VMEM((_DC,), i32),
            pltpu.VMEM((_DC,), i32),
            pltpu.VMEM((_DC // 2,), i32),
            pltpu.VMEM((_DC // 2,), i32),
            pltpu.VMEM((16,), i32),
            pltpu.VMEM((16, 16), i32),
            pltpu.VMEM((WMAX,), i32),
            pltpu.VMEM((WMAX,), i32),
            pltpu.VMEM((WMAX,), i32),
            pltpu.VMEM((WMAX,), i32),
            pltpu.VMEM((WMAX,), i32),
            pltpu.SMEM((1024,), i32),
        ],
    )
    def dk(e_hbm, pos_hbm, pose_hbm, poso_hbm,
           wt_hbm, we_hbm, wlo_hbm, whi_hbm, wf_hbm, cnt_pub,
           ec, posc, pose_v, poso_v, cntv, ctab,
           wt_s, we_s, wlo_s, whi_s, wf_s, smem):
        cid = lax.axis_index("c")
        sid = lax.axis_index("s")

        lane = lax.iota(i32, 16)
        base = sid * _DC

        @pl.when(cid == 0)
        def _():
            pltpu.sync_copy(e_hbm.at[pl.ds(base, _DC)], ec)

            for b in range(16):
                smem[b] = i32(0)

            def ha(i, c):
                v = ec[pl.ds(i * 16, 16)]
                for l in range(16):
                    e_s = v[l]
                    smem[e_s] = smem[e_s] + 1
                return c

            lax.fori_loop(0, _DV, ha, i32(0))
            cv = jnp.zeros((16,), i32)
            for b in range(16):
                cv = jnp.where(lane == i32(b), smem[b], cv)
            cntv[...] = cv
            pltpu.sync_copy(cntv, cnt_pub.at[sid])

        plsc.subcore_barrier()

        @pl.when(cid == 0)
        def _():
            pltpu.sync_copy(cnt_pub, ctab)
            tot = [i32(0)] * 16
            pre = [i32(0)] * 16
            for w in range(16):
                row = ctab[w, :]
                before = (i32(w) < sid).astype(i32)
                for b in range(16):
                    c = row[b]
                    tot[b] = tot[b] + c
                    pre[b] = pre[b] + c * before
            run = i32(0)
            for b in range(16):
                smem[64 + b] = run
                smem[32 + b] = tot[b]
                smem[b] = run + pre[b]
                run = run + tot[b]

            def hc(g, c):
                v0 = ec[pl.ds(g * 32, 16)]
                v1 = ec[pl.ds(g * 32 + 16, 16)]
                pv0 = jnp.zeros((16,), i32)
                pv1 = jnp.zeros((16,), i32)
                pe = jnp.zeros((16,), i32)
                po = jnp.zeros((16,), i32)
                for l in range(32):
                    e_s = v0[l] if l < 16 else v1[l - 16]
                    p = smem[e_s]
                    smem[e_s] = p + 1
                    if l < 16:
                        pv0 = jnp.where(lane == i32(l), p, pv0)
                    else:
                        pv1 = jnp.where(lane == i32(l - 16), p, pv1)
                    if l % 2 == 0:
                        pe = jnp.where(lane == i32(l // 2), p, pe)
                    else:
                        po = jnp.where(lane == i32(l // 2), p, po)
                posc[pl.ds(g * 32, 16)] = pv0
                posc[pl.ds(g * 32 + 16, 16)] = pv1
                pose_v[pl.ds(g * 16, 16)] = pe
                poso_v[pl.ds(g * 16, 16)] = po
                return c

            lax.fori_loop(0, _DC // 32, hc, i32(0))
            pltpu.sync_copy(posc, pos_hbm.at[pl.ds(base, _DC)])
            pltpu.sync_copy(pose_v, pose_hbm.at[pl.ds(sid * (_DC // 2), _DC // 2)])
            pltpu.sync_copy(poso_v, poso_hbm.at[pl.ds(sid * (_DC // 2), _DC // 2)])

            @pl.when(sid == 0)
            def _():
                for w in range(WMAX):
                    smem[128 + w] = i32(NT - 1)
                    smem[208 + w] = i32(0)
                    smem[288 + w] = i32(0)
                    smem[368 + w] = i32(0)
                    smem[448 + w] = i32(0)
                wcnt = i32(0)
                for e in range(16):
                    off = smem[64 + e]
                    t_ = smem[32 + e]
                    end_ = off + t_
                    t0 = lax.shift_right_logical(off, 8)
                    t1 = jnp.where(t_ > 0,
                                   lax.shift_right_logical(end_ + (TM - 1), 8),
                                   t0)

                    def wb(t, wc, _e=e, _off=off, _end=end_):
                        lo = jnp.maximum(_off, t * TM)
                        hi = jnp.minimum(_end, (t + 1) * TM)
                        fi = (lo == t * TM).astype(i32)
                        smem[128 + wc] = t
                        smem[208 + wc] = i32(_e)
                        smem[288 + wc] = lo
                        smem[368 + wc] = hi
                        smem[448 + wc] = fi
                        return wc + 1

                    wcnt = lax.fori_loop(t0, t1, wb, wcnt)
                for refs, sbase in ((wt_s, 128), (we_s, 208), (wlo_s, 288),
                                    (whi_s, 368), (wf_s, 448)):
                    for j in range(WMAX // 16):
                        v = jnp.zeros((16,), i32)
                        for l in range(16):
                            v = jnp.where(lane == i32(l),
                                          smem[sbase + j * 16 + l], v)
                        refs[pl.ds(j * 16, 16)] = v
                pltpu.sync_copy(wt_s, wt_hbm)
                pltpu.sync_copy(we_s, we_hbm)
                pltpu.sync_copy(wlo_s, wlo_hbm)
                pltpu.sync_copy(whi_s, whi_hbm)
                pltpu.sync_copy(wf_s, wf_hbm)

    p, pe_, po_, wt_, we_, wlo_, whi_, wf_, _ = dk(e_flat)
    return p, pe_, po_, (wt_, we_, wlo_, whi_, wf_)


def _scatter_x(x, pose2, poso2):
    mesh = plsc.VectorSubcoreMesh(core_axis_name="c", subcore_axis_name="s")
    rows_w = N // _NW
    chunk = 32
    n_chunks = rows_w // chunk

    @functools.partial(
        pl.kernel, mesh=mesh,
        out_type=jax.ShapeDtypeStruct((M, DIN), jnp.float32),
        scratch_types=[
            pltpu.VMEM((chunk, DIN), jnp.float32),
            pltpu.VMEM((n_chunks, chunk), jnp.int32),
            pltpu.VMEM((n_chunks, chunk), jnp.int32),
            pltpu.SemaphoreType.DMA,
        ],
    )
    def sk(x_hbm, pe_hbm, po_hbm, xs_hbm, xbuf, pev, pov, sem):
        wid = lax.axis_index("s") * _NC + lax.axis_index("c")
        pltpu.sync_copy(pe_hbm.at[pl.ds(wid * n_chunks, n_chunks)], pev)
        pltpu.sync_copy(po_hbm.at[pl.ds(wid * n_chunks, n_chunks)], pov)

        def body(ci, _):
            trow = wid * rows_w + ci * chunk
            pltpu.sync_copy(x_hbm.at[pl.ds(trow, chunk)], xbuf)
            pltpu.async_copy(xbuf, xs_hbm.at[pev.at[ci]], sem).wait()
            pltpu.async_copy(xbuf, xs_hbm.at[pov.at[ci]], sem).wait()
            return 0

        lax.fori_loop(0, n_chunks, body, 0)

    return sk(x, pose2, poso2)


def _dispatch_jnp(e_flat):
    order = jnp.argsort(e_flat, stable=True).astype(jnp.int32)
    pos = jnp.argsort(order).astype(jnp.int32)
    tid_sorted = order // K
    counts = jnp.bincount(e_flat, length=E).astype(jnp.int32)
    ends = jnp.cumsum(counts)
    offs = ends - counts
    t0 = offs // TM
    t1 = jnp.where(counts > 0, (ends + TM - 1) // TM, t0)
    nt = t1 - t0
    base = jnp.cumsum(nt) - nt
    wl_tile = jnp.full((WMAX,), NT - 1, jnp.int32)
    wl_ex = jnp.zeros((WMAX,), jnp.int32)
    wl_lo = jnp.zeros((WMAX,), jnp.int32)
    wl_hi = jnp.zeros((WMAX,), jnp.int32)
    wl_first = jnp.zeros((WMAX,), jnp.int32)
    for s in range(NT):
        m = s < nt
        widx = jnp.where(m, base + s, WMAX - 1)
        tile = t0 + s
        lo = jnp.maximum(offs, tile * TM)
        hi = jnp.minimum(ends, (tile + 1) * TM)
        wl_tile = wl_tile.at[widx].set(jnp.where(m, tile, wl_tile[widx]))
        wl_ex = wl_ex.at[widx].set(jnp.where(m, jnp.arange(E, dtype=jnp.int32), wl_ex[widx]))
        wl_lo = wl_lo.at[widx].set(jnp.where(m, lo, wl_lo[widx]))
        wl_hi = wl_hi.at[widx].set(jnp.where(m, hi, wl_hi[widx]))
        wl_first = wl_first.at[widx].set(jnp.where(m, (lo == tile * TM).astype(jnp.int32), wl_first[widx]))
    return tid_sorted, pos, (wl_tile, wl_ex, wl_lo, wl_hi, wl_first)


def kernel(x, w_gate, w_noise, expert_w, expert_b):
    idx, gates = _router(x, w_gate)
    e_flat = idx.reshape(M)
    pos, pose, poso, wl = _dispatch_sc(e_flat)
    x_sorted = _scatter_x(x, pose.reshape(N // 32, 32), poso.reshape(N // 32, 32))
    y = _gmm(*wl, x_sorted, expert_w, expert_b.reshape(E, 1, DHID))
    z = _make_sc_gather(M, DHID, M, 16)(y, pos).reshape(N, K, DHID)
    return _combine(z, gates)


# speedup vs baseline: 3.1550x; 1.0183x over previous
import functools

import jax
import jax.numpy as jnp
from jax import lax
from jax.experimental import pallas as pl
from jax.experimental.pallas import tpu as pltpu
from jax.experimental.pallas import tpu_sc as plsc

E = 16
K = 2
DIN = 1024
DHID = 4096
N = 8192
M = N * K

TM = 256
NT = M // TM
WMAX = NT + E
TN = 4096
NB = DHID // TN

_NEG = -3e38


def _router_body(x_ref, wg_ref, idx_ref, gate_ref):
    logits = jnp.dot(x_ref[...], wg_ref[...], preferred_element_type=jnp.float32)
    col = lax.broadcasted_iota(jnp.int32, logits.shape, 1)
    m1 = jnp.max(logits, axis=1, keepdims=True)
    i1 = jnp.min(jnp.where(logits == m1, col, E), axis=1, keepdims=True)
    l2 = jnp.where(col == i1, _NEG, logits)
    m2 = jnp.max(l2, axis=1, keepdims=True)
    i2 = jnp.min(jnp.where(l2 == m2, col, E), axis=1, keepdims=True)
    a = jnp.exp(m2 - m1)
    g1 = 1.0 / (1.0 + a)
    idx_ref[...] = jnp.concatenate([i1, i2], axis=1)
    gate_ref[...] = jnp.concatenate([g1, 1.0 - g1], axis=1)


def _router(x, w_gate):
    bm = 512
    return pl.pallas_call(
        _router_body,
        grid=(N // bm,),
        in_specs=[
            pl.BlockSpec((bm, DIN), lambda i: (i, 0)),
            pl.BlockSpec((DIN, E), lambda i: (0, 0)),
        ],
        out_specs=[
            pl.BlockSpec((bm, K), lambda i: (i, 0)),
            pl.BlockSpec((bm, K), lambda i: (i, 0)),
        ],
        out_shape=[
            jax.ShapeDtypeStruct((N, K), jnp.int32),
            jax.ShapeDtypeStruct((N, K), jnp.float32),
        ],
    )(x, w_gate)


def _gmm_body(tile_ref, ex_ref, lo_ref, hi_ref, first_ref,
              xs_ref, w_ref, b_ref, y_ref):
    w = pl.program_id(1)
    lo = lo_ref[w]
    hi = hi_ref[w]

    @pl.when(hi > lo)
    def _():
        acc = jnp.dot(xs_ref[...], w_ref[0], preferred_element_type=jnp.float32)
        acc += b_ref[0]
        rows = tile_ref[w] * TM + lax.broadcasted_iota(jnp.int32, (TM, 1), 0)
        mask = (rows >= lo) & (rows < hi)
        contrib = jnp.where(mask, acc, 0.0)
        y_ref[...] = jnp.where(first_ref[w] > 0, contrib, contrib + y_ref[...])


def _gmm(wl_tile, wl_ex, wl_lo, wl_hi, wl_first, x_sorted, expert_w, expert_b):
    grid_spec = pltpu.PrefetchScalarGridSpec(
        num_scalar_prefetch=5,
        grid=(NB, WMAX),
        in_specs=[
            pl.BlockSpec((TM, DIN), lambda n, w, t, e, lo, hi, f: (t[w], 0)),
            pl.BlockSpec((1, DIN, TN), lambda n, w, t, e, lo, hi, f: (e[w], 0, n)),
            pl.BlockSpec((1, 1, TN), lambda n, w, t, e, lo, hi, f: (e[w], 0, n)),
        ],
        out_specs=pl.BlockSpec((TM, TN), lambda n, w, t, e, lo, hi, f: (t[w], n)),
    )
    return pl.pallas_call(
        _gmm_body,
        grid_spec=grid_spec,
        out_shape=jax.ShapeDtypeStruct((M, DHID), jnp.float32),
        compiler_params=pltpu.CompilerParams(
            dimension_semantics=("arbitrary", "arbitrary")),
    )(wl_tile, wl_ex, wl_lo, wl_hi, wl_first, x_sorted, expert_w, expert_b)


def _combine_body(z_ref, g_ref, o_ref):
    z = z_ref[...]
    g0 = g_ref[:, 0:1]
    g1 = g_ref[:, 1:2]
    o_ref[...] = g0 * z[:, 0, :] + g1 * z[:, 1, :]


def _combine(z, gates):
    bm = 256
    return pl.pallas_call(
        _combine_body,
        grid=(N // bm,),
        in_specs=[
            pl.BlockSpec((bm, K, DHID), lambda i: (i, 0, 0)),
            pl.BlockSpec((bm, K), lambda i: (i, 0)),
        ],
        out_specs=pl.BlockSpec((bm, DHID), lambda i: (i, 0)),
        out_shape=jax.ShapeDtypeStruct((N, DHID), jnp.float32),
    )(z, gates)


_NC = 2
_NS = 16
_NW = _NC * _NS


def _make_sc_gather(v, d, b, chunk, dtype=jnp.float32):
    per_w = b // _NW
    n_chunks = per_w // chunk
    mesh = plsc.VectorSubcoreMesh(core_axis_name="c", subcore_axis_name="s")

    @functools.partial(
        pl.kernel, mesh=mesh,
        out_type=jax.ShapeDtypeStruct((b, d), dtype),
        scratch_types=[
            pltpu.VMEM((n_chunks, chunk), jnp.int32),
            pltpu.VMEM((chunk, d), dtype),
            pltpu.VMEM((chunk, d), dtype),
            pltpu.SemaphoreType.DMA,
            pltpu.SemaphoreType.DMA,
        ],
    )
    def gk(table_hbm, idx_hbm, out_hbm, idxv, rows0, rows1, sem0, sem1):
        wid = lax.axis_index("s") * _NC + lax.axis_index("c")
        pltpu.sync_copy(idx_hbm.at[pl.ds(wid * n_chunks, n_chunks)], idxv)
        bufs = (rows0, rows1)
        sems = (sem0, sem1)

        def start(ci, buf, sem):
            pltpu.async_copy(table_hbm.at[idxv.at[ci]], buf, sem)

        def drain(buf, sem):
            pltpu.make_async_copy(table_hbm.at[pl.ds(0, chunk)], buf, sem).wait()

        start(0, bufs[0], sems[0])
        for ci in range(n_chunks):
            buf, sem = bufs[ci % 2], sems[ci % 2]
            if ci + 1 < n_chunks:
                start(ci + 1, bufs[(ci + 1) % 2], sems[(ci + 1) % 2])
            drain(buf, sem)
            pltpu.sync_copy(
                buf, out_hbm.at[pl.ds(wid * per_w + ci * chunk, chunk)])

    return gk


_DC = M // _NS
_DV = _DC // 16


def _dispatch_sc(e_flat):
    mesh = plsc.VectorSubcoreMesh(core_axis_name="c", subcore_axis_name="s")
    i32 = jnp.int32

    @functools.partial(
        pl.kernel, mesh=mesh,
        out_type=[
            jax.ShapeDtypeStruct((M,), i32),
            jax.ShapeDtypeStruct((N,), i32),
            jax.ShapeDtypeStruct((N,), i32),
            jax.ShapeDtypeStruct((WMAX,), i32),
            jax.ShapeDtypeStruct((WMAX,), i32),
            jax.ShapeDtypeStruct((WMAX,), i32),
            jax.ShapeDtypeStruct((WMAX,), i32),
            jax.ShapeDtypeStruct((WMAX,), i32),
            jax.ShapeDtypeStruct((_NS, 16), i32),
        ],
        scratch_types=[
            pltpu.VMEM((_DC,), i32),
            pltpu.VMEM((_DC,), i32),
            pltpu.VMEM((_DC // 2,), i32),
            pltpu.VMEM((_DC // 2,), i32),
            pltpu.VMEM((16,), i32),
            pltpu.VMEM((16, 16), i32),
            pltpu.VMEM((WMAX,), i32),
            pltpu.VMEM((WMAX,), i32),
            pltpu.VMEM((WMAX,), i32),
            pltpu.VMEM((WMAX,), i32),
            pltpu.VMEM((WMAX,), i32),
            pltpu.SMEM((1024,), i32),
        ],
    )
    def dk(e_hbm, pos_hbm, pose_hbm, poso_hbm,
           wt_hbm, we_hbm, wlo_hbm, whi_hbm, wf_hbm, cnt_pub,
           ec, posc, pose_v, poso_v, cntv, ctab,
           wt_s, we_s, wlo_s, whi_s, wf_s, smem):
        cid = lax.axis_index("c")
        sid = lax.axis_index("s")

        lane = lax.iota(i32, 16)
        base = sid * _DC

        @pl.when(cid == 0)
        def _():
            pltpu.sync_copy(e_hbm.at[pl.ds(base, _DC)], ec)

            for b in range(16):
                smem[b] = i32(0)

            def ha(i, c):
                v = ec[pl.ds(i * 16, 16)]
                for l in range(16):
                    e_s = v[l]
                    smem[e_s] = smem[e_s] + 1
                return c

            lax.fori_loop(0, _DV, ha, i32(0))
            cv = jnp.zeros((16,), i32)
            for b in range(16):
                cv = jnp.where(lane == i32(b), smem[b], cv)
            cntv[...] = cv
            pltpu.sync_copy(cntv, cnt_pub.at[sid])

        plsc.subcore_barrier()

        @pl.when(cid == 0)
        def _():
            pltpu.sync_copy(cnt_pub, ctab)
            tot = [i32(0)] * 16
            pre = [i32(0)] * 16
            for w in range(16):
                row = ctab[w, :]
                before = (i32(w) < sid).astype(i32)
                for b in range(16):
                    c = row[b]
                    tot[b] = tot[b] + c
                    pre[b] = pre[b] + c * before
            run = i32(0)
            for b in range(16):
                smem[64 + b] = run
                smem[32 + b] = tot[b]
                smem[b] = run + pre[b]
                run = run + tot[b]

            def hc(g, c):
                v0 = ec[pl.ds(g * 32, 16)]
                v1 = ec[pl.ds(g * 32 + 16, 16)]
                pv0 = jnp.zeros((16,), i32)
                pv1 = jnp.zeros((16,), i32)
                pe = jnp.zeros((16,), i32)
                po = jnp.zeros((16,), i32)
                for l in range(32):
                    e_s = v0[l] if l < 16 else v1[l - 16]
                    p = smem[e_s]
                    smem[e_s] = p + 1
                    if l < 16:
                        pv0 = jnp.where(lane == i32(l), p, pv0)
                    else:
                        pv1 = jnp.where(lane == i32(l - 16), p, pv1)
                    if l % 2 == 0:
                        pe = jnp.where(lane == i32(l // 2), p, pe)
                    else:
                        po = jnp.where(lane == i32(l // 2), p, po)
                posc[pl.ds(g * 32, 16)] = pv0
                posc[pl.ds(g * 32 + 16, 16)] = pv1
                pose_v[pl.ds(g * 16, 16)] = pe
                poso_v[pl.ds(g * 16, 16)] = po
                return c

            lax.fori_loop(0, _DC // 32, hc, i32(0))
            pltpu.sync_copy(posc, pos_hbm.at[pl.ds(base, _DC)])
            pltpu.sync_copy(pose_v, pose_hbm.at[pl.ds(sid * (_DC // 2), _DC // 2)])
            pltpu.sync_copy(poso_v, poso_hbm.at[pl.ds(sid * (_DC // 2), _DC // 2)])

            @pl.when(sid == 0)
            def _():
                for w in range(WMAX):
                    smem[128 + w] = i32(NT - 1)
                    smem[208 + w] = i32(0)
                    smem[288 + w] = i32(0)
                    smem[368 + w] = i32(0)
                    smem[448 + w] = i32(0)
                wcnt = i32(0)
                for e in range(16):
                    off = smem[64 + e]
                    t_ = smem[32 + e]
                    end_ = off + t_
                    t0 = lax.shift_right_logical(off, 8)
                    t1 = jnp.where(t_ > 0,
                                   lax.shift_right_logical(end_ + (TM - 1), 8),
                                   t0)

                    def wb(t, wc, _e=e, _off=off, _end=end_):
                        lo = jnp.maximum(_off, t * TM)
                        hi = jnp.minimum(_end, (t + 1) * TM)
                        fi = (lo == t * TM).astype(i32)
                        smem[128 + wc] = t
                        smem[208 + wc] = i32(_e)
                        smem[288 + wc] = lo
                        smem[368 + wc] = hi
                        smem[448 + wc] = fi
                        return wc + 1

                    wcnt = lax.fori_loop(t0, t1, wb, wcnt)
                for refs, sbase in ((wt_s, 128), (we_s, 208), (wlo_s, 288),
                                    (whi_s, 368), (wf_s, 448)):
                    for j in range(WMAX // 16):
                        v = jnp.zeros((16,), i32)
                        for l in range(16):
                            v = jnp.where(lane == i32(l),
                                          smem[sbase + j * 16 + l], v)
                        refs[pl.ds(j * 16, 16)] = v
                pltpu.sync_copy(wt_s, wt_hbm)
                pltpu.sync_copy(we_s, we_hbm)
                pltpu.sync_copy(wlo_s, wlo_hbm)
                pltpu.sync_copy(whi_s, whi_hbm)
                pltpu.sync_copy(wf_s, wf_hbm)

    p, pe_, po_, wt_, we_, wlo_, whi_, wf_, _ = dk(e_flat)
    return p, pe_, po_, (wt_, we_, wlo_, whi_, wf_)


def _scatter_x(x, pose2, poso2):
    mesh = plsc.VectorSubcoreMesh(core_axis_name="c", subcore_axis_name="s")
    rows_w = N // _NW
    chunk = 32
    n_chunks = rows_w // chunk

    @functools.partial(
        pl.kernel, mesh=mesh,
        out_type=jax.ShapeDtypeStruct((M, DIN), jnp.float32),
        scratch_types=[
            pltpu.VMEM((chunk, DIN), jnp.float32),
            pltpu.VMEM((n_chunks, chunk), jnp.int32),
            pltpu.VMEM((n_chunks, chunk), jnp.int32),
            pltpu.SemaphoreType.DMA,
        ],
    )
    def sk(x_hbm, pe_hbm, po_hbm, xs_hbm, xbuf, pev, pov, sem):
        wid = lax.axis_index("s") * _NC + lax.axis_index("c")
        pltpu.sync_copy(pe_hbm.at[pl.ds(wid * n_chunks, n_chunks)], pev)
        pltpu.sync_copy(po_hbm.at[pl.ds(wid * n_chunks, n_chunks)], pov)

        def body(ci, _):
            trow = wid * rows_w + ci * chunk
            pltpu.sync_copy(x_hbm.at[pl.ds(trow, chunk)], xbuf)
            pltpu.async_copy(xbuf, xs_hbm.at[pev.at[ci]], sem).wait()
            pltpu.async_copy(xbuf, xs_hbm.at[pov.at[ci]], sem).wait()
            return 0

        lax.fori_loop(0, n_chunks, body, 0)

    return sk(x, pose2, poso2)


def _dispatch_jnp(e_flat):
    order = jnp.argsort(e_flat, stable=True).astype(jnp.int32)
    pos = jnp.argsort(order).astype(jnp.int32)
    tid_sorted = order // K
    counts = jnp.bincount(e_flat, length=E).astype(jnp.int32)
    ends = jnp.cumsum(counts)
    offs = ends - counts
    t0 = offs // TM
    t1 = jnp.where(counts > 0, (ends + TM - 1) // TM, t0)
    nt = t1 - t0
    base = jnp.cumsum(nt) - nt
    wl_tile = jnp.full((WMAX,), NT - 1, jnp.int32)
    wl_ex = jnp.zeros((WMAX,), jnp.int32)
    wl_lo = jnp.zeros((WMAX,), jnp.int32)
    wl_hi = jnp.zeros((WMAX,), jnp.int32)
    wl_first = jnp.zeros((WMAX,), jnp.int32)
    for s in range(NT):
        m = s < nt
        widx = jnp.where(m, base + s, WMAX - 1)
        tile = t0 + s
        lo = jnp.maximum(offs, tile * TM)
        hi = jnp.minimum(ends, (tile + 1) * TM)
        wl_tile = wl_tile.at[widx].set(jnp.where(m, tile, wl_tile[widx]))
        wl_ex = wl_ex.at[widx].set(jnp.where(m, jnp.arange(E, dtype=jnp.int32), wl_ex[widx]))
        wl_lo = wl_lo.at[widx].set(jnp.where(m, lo, wl_lo[widx]))
        wl_hi = wl_hi.at[widx].set(jnp.where(m, hi, wl_hi[widx]))
        wl_first = wl_first.at[widx].set(jnp.where(m, (lo == tile * TM).astype(jnp.int32), wl_first[widx]))
    return tid_sorted, pos, (wl_tile, wl_ex, wl_lo, wl_hi, wl_first)


def kernel(x, w_gate, w_noise, expert_w, expert_b):
    idx, gates = _router(x, w_gate)
    e_flat = idx.reshape(M)
    pos, pose, poso, wl = _dispatch_sc(e_flat)
    x_sorted = _scatter_x(x, pose.reshape(N // 32, 32), poso.reshape(N // 32, 32))
    y = _gmm(*wl, x_sorted, expert_w, expert_b.reshape(E, 1, DHID))
    z = _make_sc_gather(M, DHID, M, 8)(y, pos.reshape(M // 8, 8)).reshape(N, K, DHID)
    return _combine(z, gates)


# speedup vs baseline: 3.1658x; 1.0034x over previous
import functools

import jax
import jax.numpy as jnp
from jax import lax
from jax.experimental import pallas as pl
from jax.experimental.pallas import tpu as pltpu
from jax.experimental.pallas import tpu_sc as plsc

E = 16
K = 2
DIN = 1024
DHID = 4096
N = 8192
M = N * K

TM = 256
NT = M // TM
WMAX = NT + E
TN = 4096
NB = DHID // TN

_NEG = -3e38


def _router_body(x_ref, wg_ref, idx_ref, gate_ref):
    logits = jnp.dot(x_ref[...], wg_ref[...], preferred_element_type=jnp.float32)
    col = lax.broadcasted_iota(jnp.int32, logits.shape, 1)
    m1 = jnp.max(logits, axis=1, keepdims=True)
    i1 = jnp.min(jnp.where(logits == m1, col, E), axis=1, keepdims=True)
    l2 = jnp.where(col == i1, _NEG, logits)
    m2 = jnp.max(l2, axis=1, keepdims=True)
    i2 = jnp.min(jnp.where(l2 == m2, col, E), axis=1, keepdims=True)
    a = jnp.exp(m2 - m1)
    g1 = 1.0 / (1.0 + a)
    idx_ref[...] = jnp.concatenate([i1, i2], axis=1)
    gate_ref[...] = jnp.concatenate([g1, 1.0 - g1], axis=1)


def _router(x, w_gate):
    bm = 512
    return pl.pallas_call(
        _router_body,
        grid=(N // bm,),
        in_specs=[
            pl.BlockSpec((bm, DIN), lambda i: (i, 0)),
            pl.BlockSpec((DIN, E), lambda i: (0, 0)),
        ],
        out_specs=[
            pl.BlockSpec((bm, K), lambda i: (i, 0)),
            pl.BlockSpec((bm, K), lambda i: (i, 0)),
        ],
        out_shape=[
            jax.ShapeDtypeStruct((N, K), jnp.int32),
            jax.ShapeDtypeStruct((N, K), jnp.float32),
        ],
    )(x, w_gate)


def _gmm_body(tile_ref, ex_ref, lo_ref, hi_ref, first_ref,
              xs_ref, w_ref, b_ref, y_ref):
    w = pl.program_id(1)
    lo = lo_ref[w]
    hi = hi_ref[w]

    @pl.when(hi > lo)
    def _():
        acc = jnp.dot(xs_ref[...], w_ref[0], preferred_element_type=jnp.float32)
        acc += b_ref[0]
        rows = tile_ref[w] * TM + lax.broadcasted_iota(jnp.int32, (TM, 1), 0)
        mask = (rows >= lo) & (rows < hi)
        contrib = jnp.where(mask, acc, 0.0)
        y_ref[...] = jnp.where(first_ref[w] > 0, contrib, contrib + y_ref[...])


def _gmm(wl_tile, wl_ex, wl_lo, wl_hi, wl_first, x_sorted, expert_w, expert_b):
    grid_spec = pltpu.PrefetchScalarGridSpec(
        num_scalar_prefetch=5,
        grid=(NB, WMAX),
        in_specs=[
            pl.BlockSpec((TM, DIN), lambda n, w, t, e, lo, hi, f: (t[w], 0)),
            pl.BlockSpec((1, DIN, TN), lambda n, w, t, e, lo, hi, f: (e[w], 0, n)),
            pl.BlockSpec((1, 1, TN), lambda n, w, t, e, lo, hi, f: (e[w], 0, n)),
        ],
        out_specs=pl.BlockSpec((TM, TN), lambda n, w, t, e, lo, hi, f: (t[w], n)),
    )
    return pl.pallas_call(
        _gmm_body,
        grid_spec=grid_spec,
        out_shape=jax.ShapeDtypeStruct((M, DHID), jnp.float32),
        compiler_params=pltpu.CompilerParams(
            dimension_semantics=("arbitrary", "arbitrary")),
    )(wl_tile, wl_ex, wl_lo, wl_hi, wl_first, x_sorted, expert_w, expert_b)


def _combine_body(z_ref, g_ref, o_ref):
    z = z_ref[...]
    g0 = g_ref[:, 0:1]
    g1 = g_ref[:, 1:2]
    o_ref[...] = g0 * z[:, 0, :] + g1 * z[:, 1, :]


def _combine(z, gates):
    bm = 256
    return pl.pallas_call(
        _combine_body,
        grid=(N // bm,),
        in_specs=[
            pl.BlockSpec((bm, K, DHID), lambda i: (i, 0, 0)),
            pl.BlockSpec((bm, K), lambda i: (i, 0)),
        ],
        out_specs=pl.BlockSpec((bm, DHID), lambda i: (i, 0)),
        out_shape=jax.ShapeDtypeStruct((N, DHID), jnp.float32),
    )(z, gates)


_NC = 2
_NS = 16
_NW = _NC * _NS


def _make_sc_gather(v, d, b, chunk, dtype=jnp.float32):
    per_w = b // _NW
    n_chunks = per_w // chunk
    mesh = plsc.VectorSubcoreMesh(core_axis_name="c", subcore_axis_name="s")

    @functools.partial(
        pl.kernel, mesh=mesh,
        out_type=jax.ShapeDtypeStruct((b, d), dtype),
        scratch_types=[
            pltpu.VMEM((n_chunks, chunk), jnp.int32),
            pltpu.VMEM((chunk, d), dtype),
            pltpu.VMEM((chunk, d), dtype),
            pltpu.SemaphoreType.DMA,
            pltpu.SemaphoreType.DMA,
        ],
    )
    def gk(table_hbm, idx_hbm, out_hbm, idxv, rows0, rows1, sem0, sem1):
        wid = lax.axis_index("s") * _NC + lax.axis_index("c")
        pltpu.sync_copy(idx_hbm.at[pl.ds(wid * n_chunks, n_chunks)], idxv)
        bufs = (rows0, rows1)
        sems = (sem0, sem1)

        def start(ci, buf, sem):
            pltpu.async_copy(table_hbm.at[idxv.at[ci]], buf, sem)

        def drain(buf, sem):
            pltpu.make_async_copy(table_hbm.at[pl.ds(0, chunk)], buf, sem).wait()

        start(0, bufs[0], sems[0])
        for ci in range(n_chunks):
            buf, sem = bufs[ci % 2], sems[ci % 2]
            if ci + 1 < n_chunks:
                start(ci + 1, bufs[(ci + 1) % 2], sems[(ci + 1) % 2])
            drain(buf, sem)
            pltpu.sync_copy(
                buf, out_hbm.at[pl.ds(wid * per_w + ci * chunk, chunk)])

    return gk


_DC = M // _NS
_DV = _DC // 16


def _dispatch_sc(e_flat):
    mesh = plsc.VectorSubcoreMesh(core_axis_name="c", subcore_axis_name="s")
    i32 = jnp.int32

    @functools.partial(
        pl.kernel, mesh=mesh,
        out_type=[
            jax.ShapeDtypeStruct((M,), i32),
            jax.ShapeDtypeStruct((N,), i32),
            jax.ShapeDtypeStruct((N,), i32),
            jax.ShapeDtypeStruct((WMAX,), i32),
            jax.ShapeDtypeStruct((WMAX,), i32),
            jax.ShapeDtypeStruct((WMAX,), i32),
            jax.ShapeDtypeStruct((WMAX,), i32),
            jax.ShapeDtypeStruct((WMAX,), i32),
            jax.ShapeDtypeStruct((_NS, 16), i32),
        ],
        scratch_types=[
            pltpu.VMEM((_DC,), i32),
            pltpu.VMEM((_DC,), i32),
            pltpu.VMEM((_DC // 2,), i32),
            pltpu.VMEM((_DC // 2,), i32),
            pltpu.VMEM((16,), i32),
            pltpu.VMEM((16, 16), i32),
            pltpu.VMEM((WMAX,), i32),
            pltpu.VMEM((WMAX,), i32),
            pltpu.VMEM((WMAX,), i32),
            pltpu.VMEM((WMAX,), i32),
            pltpu.VMEM((WMAX,), i32),
            pltpu.SMEM((1024,), i32),
        ],
    )
    def dk(e_hbm, pos_hbm, pose_hbm, poso_hbm,
           wt_hbm, we_hbm, wlo_hbm, whi_hbm, wf_hbm, cnt_pub,
           ec, posc, pose_v, poso_v, cntv, ctab,
           wt_s, we_s, wlo_s, whi_s, wf_s, smem):
        cid = lax.axis_index("c")
        sid = lax.axis_index("s")

        lane = lax.iota(i32, 16)
        base = sid * _DC

        @pl.when(cid == 0)
        def _():
            pltpu.sync_copy(e_hbm.at[pl.ds(base, _DC)], ec)

            for b in range(16):
                smem[b] = i32(0)

            def ha(i, c):
                v = ec[pl.ds(i * 16, 16)]
                for l in range(16):
                    e_s = v[l]
                    smem[e_s] = smem[e_s] + 1
                return c

            lax.fori_loop(0, _DV, ha, i32(0))
            cv = jnp.zeros((16,), i32)
            for b in range(16):
                cv = jnp.where(lane == i32(b), smem[b], cv)
            cntv[...] = cv
            pltpu.sync_copy(cntv, cnt_pub.at[sid])

        plsc.subcore_barrier()

        @pl.when(cid == 0)
        def _():
            pltpu.sync_copy(cnt_pub, ctab)
            tot = [i32(0)] * 16
            pre = [i32(0)] * 16
            for w in range(16):
                row = ctab[w, :]
                before = (i32(w) < sid).astype(i32)
                for b in range(16):
                    c = row[b]
                    tot[b] = tot[b] + c
                    pre[b] = pre[b] + c * before
            run = i32(0)
            for b in range(16):
                smem[64 + b] = run
                smem[32 + b] = tot[b]
                smem[b] = run + pre[b]
                run = run + tot[b]

            def hc(g, c):
                v0 = ec[pl.ds(g * 32, 16)]
                v1 = ec[pl.ds(g * 32 + 16, 16)]
                pv0 = jnp.zeros((16,), i32)
                pv1 = jnp.zeros((16,), i32)
                pe = jnp.zeros((16,), i32)
                po = jnp.zeros((16,), i32)
                for l in range(32):
                    e_s = v0[l] if l < 16 else v1[l - 16]
                    p = smem[e_s]
                    smem[e_s] = p + 1
                    if l < 16:
                        pv0 = jnp.where(lane == i32(l), p, pv0)
                    else:
                        pv1 = jnp.where(lane == i32(l - 16), p, pv1)
                    if l % 2 == 0:
                        pe = jnp.where(lane == i32(l // 2), p, pe)
                    else:
                        po = jnp.where(lane == i32(l // 2), p, po)
                posc[pl.ds(g * 32, 16)] = pv0
                posc[pl.ds(g * 32 + 16, 16)] = pv1
                pose_v[pl.ds(g * 16, 16)] = pe
                poso_v[pl.ds(g * 16, 16)] = po
                return c

            lax.fori_loop(0, _DC // 32, hc, i32(0))
            pltpu.sync_copy(posc, pos_hbm.at[pl.ds(base, _DC)])
            pltpu.sync_copy(pose_v, pose_hbm.at[pl.ds(sid * (_DC // 2), _DC // 2)])
            pltpu.sync_copy(poso_v, poso_hbm.at[pl.ds(sid * (_DC // 2), _DC // 2)])

            @pl.when(sid == 0)
            def _():
                for w in range(WMAX):
                    smem[128 + w] = i32(NT - 1)
                    smem[208 + w] = i32(0)
                    smem[288 + w] = i32(0)
                    smem[368 + w] = i32(0)
                    smem[448 + w] = i32(0)
                wcnt = i32(0)
                for e in range(16):
                    off = smem[64 + e]
                    t_ = smem[32 + e]
                    end_ = off + t_
                    t0 = lax.shift_right_logical(off, 8)
                    t1 = jnp.where(t_ > 0,
                                   lax.shift_right_logical(end_ + (TM - 1), 8),
                                   t0)

                    def wb(t, wc, _e=e, _off=off, _end=end_):
                        lo = jnp.maximum(_off, t * TM)
                        hi = jnp.minimum(_end, (t + 1) * TM)
                        fi = (lo == t * TM).astype(i32)
                        smem[128 + wc] = t
                        smem[208 + wc] = i32(_e)
                        smem[288 + wc] = lo
                        smem[368 + wc] = hi
                        smem[448 + wc] = fi
                        return wc + 1

                    wcnt = lax.fori_loop(t0, t1, wb, wcnt)
                for refs, sbase in ((wt_s, 128), (we_s, 208), (wlo_s, 288),
                                    (whi_s, 368), (wf_s, 448)):
                    for j in range(WMAX // 16):
                        v = jnp.zeros((16,), i32)
                        for l in range(16):
                            v = jnp.where(lane == i32(l),
                                          smem[sbase + j * 16 + l], v)
                        refs[pl.ds(j * 16, 16)] = v
                pltpu.sync_copy(wt_s, wt_hbm)
                pltpu.sync_copy(we_s, we_hbm)
                pltpu.sync_copy(wlo_s, wlo_hbm)
                pltpu.sync_copy(whi_s, whi_hbm)
                pltpu.sync_copy(wf_s, wf_hbm)

    p, pe_, po_, wt_, we_, wlo_, whi_, wf_, _ = dk(e_flat)
    return p, pe_, po_, (wt_, we_, wlo_, whi_, wf_)


def _scatter_x(x, pose2, poso2):
    mesh = plsc.VectorSubcoreMesh(core_axis_name="c", subcore_axis_name="s")
    rows_w = N // _NW
    chunk = 32
    n_chunks = rows_w // chunk

    @functools.partial(
        pl.kernel, mesh=mesh,
        out_type=jax.ShapeDtypeStruct((M, DIN), jnp.float32),
        scratch_types=[
            pltpu.VMEM((chunk, DIN), jnp.float32),
            pltpu.VMEM((chunk, DIN), jnp.float32),
            pltpu.VMEM((n_chunks, chunk), jnp.int32),
            pltpu.VMEM((n_chunks, chunk), jnp.int32),
            pltpu.SemaphoreType.DMA,
            pltpu.SemaphoreType.DMA,
            pltpu.SemaphoreType.DMA,
            pltpu.SemaphoreType.DMA,
        ],
    )
    def sk(x_hbm, pe_hbm, po_hbm, xs_hbm, b0, b1, pev, pov, r0, r1, w0, w1):
        wid = lax.axis_index("s") * _NC + lax.axis_index("c")
        pltpu.sync_copy(pe_hbm.at[pl.ds(wid * n_chunks, n_chunks)], pev)
        pltpu.sync_copy(po_hbm.at[pl.ds(wid * n_chunks, n_chunks)], pov)
        bufs = (b0, b1)
        rsems = (r0, r1)
        wsems = (w0, w1)

        def rd(ci, buf, sem):
            pltpu.async_copy(
                x_hbm.at[pl.ds(wid * rows_w + ci * chunk, chunk)], buf, sem)

        def rdwait(buf, sem):
            pltpu.make_async_copy(x_hbm.at[pl.ds(0, chunk)], buf, sem).wait()

        def sc(ci, buf, sem):
            pltpu.async_copy(buf, xs_hbm.at[pev.at[ci]], sem)
            pltpu.async_copy(buf, xs_hbm.at[pov.at[ci]], sem)

        def scwait(buf, sem):
            pltpu.make_async_copy(buf, xs_hbm.at[pl.ds(0, chunk)], sem).wait()
            pltpu.make_async_copy(buf, xs_hbm.at[pl.ds(0, chunk)], sem).wait()

        rd(0, bufs[0], rsems[0])
        for ci in range(n_chunks):
            cur = ci % 2
            nxt = (ci + 1) % 2
            rdwait(bufs[cur], rsems[cur])
            if ci + 1 < n_chunks:
                if ci >= 1:
                    scwait(bufs[nxt], wsems[nxt])
                rd(ci + 1, bufs[nxt], rsems[nxt])
            sc(ci, bufs[cur], wsems[cur])
        scwait(bufs[(n_chunks - 1) % 2], wsems[(n_chunks - 1) % 2])
        if n_chunks >= 2:
            scwait(bufs[n_chunks % 2], wsems[n_chunks % 2])

    return sk(x, pose2, poso2)


def _dispatch_jnp(e_flat):
    order = jnp.argsort(e_flat, stable=True).astype(jnp.int32)
    pos = jnp.argsort(order).astype(jnp.int32)
    tid_sorted = order // K
    counts = jnp.bincount(e_flat, length=E).astype(jnp.int32)
    ends = jnp.cumsum(counts)
    offs = ends - counts
    t0 = offs // TM
    t1 = jnp.where(counts > 0, (ends + TM - 1) // TM, t0)
    nt = t1 - t0
    base = jnp.cumsum(nt) - nt
    wl_tile = jnp.full((WMAX,), NT - 1, jnp.int32)
    wl_ex = jnp.zeros((WMAX,), jnp.int32)
    wl_lo = jnp.zeros((WMAX,), jnp.int32)
    wl_hi = jnp.zeros((WMAX,), jnp.int32)
    wl_first = jnp.zeros((WMAX,), jnp.int32)
    for s in range(NT):
        m = s < nt
        widx = jnp.where(m, base + s, WMAX - 1)
        tile = t0 + s
        lo = jnp.maximum(offs, tile * TM)
        hi = jnp.minimum(ends, (tile + 1) * TM)
        wl_tile = wl_tile.at[widx].set(jnp.where(m, tile, wl_tile[widx]))
        wl_ex = wl_ex.at[widx].set(jnp.where(m, jnp.arange(E, dtype=jnp.int32), wl_ex[widx]))
        wl_lo = wl_lo.at[widx].set(jnp.where(m, lo, wl_lo[widx]))
        wl_hi = wl_hi.at[widx].set(jnp.where(m, hi, wl_hi[widx]))
        wl_first = wl_first.at[widx].set(jnp.where(m, (lo == tile * TM).astype(jnp.int32), wl_first[widx]))
    return tid_sorted, pos, (wl_tile, wl_ex, wl_lo, wl_hi, wl_first)


def kernel(x, w_gate, w_noise, expert_w, expert_b):
    idx, gates = _router(x, w_gate)
    e_flat = idx.reshape(M)
    pos, pose, poso, wl = _dispatch_sc(e_flat)
    x_sorted = _scatter_x(x, pose.reshape(N // 32, 32), poso.reshape(N // 32, 32))
    y = _gmm(*wl, x_sorted, expert_w, expert_b.reshape(E, 1, DHID))
    z = _make_sc_gather(M, DHID, M, 8)(y, pos.reshape(M // 8, 8)).reshape(N, K, DHID)
    return _combine(z, gates)
